# baseline (speedup 1.0000x reference)
import math

import jax
import jax.numpy as jnp
from jax.experimental import pallas as pl

_C = math.sqrt(2.0 / math.pi)


def _gelu_block(x_ref, o_ref):
    x = x_ref[...]
    u = _C * (x + 0.044715 * (x * x * x))
    o_ref[...] = 0.5 * x * (1.0 + jnp.tanh(u))


def kernel(x, log_k_blend):
    B, T, D = x.shape
    R = B * T
    x2 = x.reshape(R, D)
    BR = 512
    out = pl.pallas_call(
        _gelu_block,
        grid=(R // BR,),
        in_specs=[pl.BlockSpec((BR, D), lambda i: (i, 0))],
        out_specs=pl.BlockSpec((BR, D), lambda i: (i, 0)),
        out_shape=jax.ShapeDtypeStruct((R, D), x.dtype),
    )(x2)
    return out.reshape(B, T, D)
